# baseline (speedup 1.0000x reference)
import dataclasses

import jax
import jax.numpy as jnp
from jax import lax
from jax.experimental import pallas as pl
from jax.experimental.pallas import tpu as pltpu
from jax.experimental.pallas import tpu_sc as plsc

N = 50000
E = 1600000
NC = 2
NS = 16
NW = NC * NS
L = 16

CHUNK = 128
NCHUNK = E // CHUNK
SSC = 32
NCHUNKP = ((NCHUNK + SSC - 1) // SSC) * SSC
NSS = NCHUNKP // SSC
SSI = (NSS + NW - 1) // NW
NACC = N + 16
ZR = 80
NZCH = N // ZR
ZITERS = (NZCH + NS - 1) // NS
CAPG = 512
CAPB = CAPG // 8
ACC4 = 144

_mesh = plsc.VectorSubcoreMesh(core_axis_name="c", subcore_axis_name="s",
                               num_cores=NC, num_subcores=NS)

_sc_params = pltpu.CompilerParams(needs_layout_passes=False,
                                  use_tc_tiling_on_sc=False)


def _zero_vmem_2d(ref, rows, dp):
    z = jnp.zeros((L,), jnp.float32)

    @pl.loop(0, rows)
    def _(r):
        for j in range(0, dp, L):
            ref[r, pl.ds(j, L)] = z


def _make_sc_agg(dp, compact, ssc, sbc):
    nss = NCHUNKP // ssc
    ssi = (nss + NW - 1) // NW
    nsb = ssc // sbc
    out_type = [jax.ShapeDtypeStruct((NC, N, dp), jnp.float32)]
    if compact:
        out_type += [
            jax.ShapeDtypeStruct((NW, CAPB, CHUNK), jnp.int32),
            jax.ShapeDtypeStruct((NW, CAPB, CHUNK), jnp.int32),
            jax.ShapeDtypeStruct((NW, L), jnp.int32),
        ]
    scratch = [
        pltpu.VMEM_SHARED((NACC, dp), jnp.float32),
        pltpu.VMEM((2, ssc, CHUNK), jnp.int32),
        pltpu.VMEM((2, ssc, CHUNK), jnp.int32),
        pltpu.VMEM((2, sbc * CHUNK, dp), jnp.float32),
        pltpu.VMEM((ZR, dp), jnp.float32),
        pltpu.SemaphoreType.DMA,
        pltpu.SemaphoreType.DMA,
    ]
    if compact:
        scratch += [
            pltpu.VMEM((CAPB, CHUNK), jnp.int32),
            pltpu.VMEM((CAPB, CHUNK), jnp.int32),
            pltpu.VMEM((L,), jnp.int32),
            pltpu.SMEM((8,), jnp.int32),
        ]

    def body(h_hbm, e_hbm, *rest):
        if compact:
            (out_hbm, ssrc_hbm, sdst_hbm, cnt_hbm,
             acc, sidx, didx, rows, zbuf, gsem, isem,
             lsrc, ldst, cbuf, cms) = rest
        else:
            out_hbm, acc, sidx, didx, rows, zbuf, gsem, isem = rest
        c = lax.axis_index("c")
        s = lax.axis_index("s")
        wid = s * NC + c

        def idx_copies(ss, p):
            base = ss * ssc
            return (
                pltpu.make_async_copy(e_hbm.at[0, pl.ds(base, ssc)],
                                      sidx.at[p], isem),
                pltpu.make_async_copy(e_hbm.at[1, pl.ds(base, ssc)],
                                      didx.at[p], isem),
            )

        def gather_copy(p, ci, rset, j):
            return pltpu.make_async_copy(
                h_hbm.at[sidx.at[p, ci]],
                rows.at[rset, pl.ds(j * CHUNK, CHUNK)], gsem)

        _zero_vmem_2d(zbuf, ZR, dp)
        if compact:
            cms[0] = 0

        s0 = wid

        @pl.when(s0 < nss)
        def _():
            for cp in idx_copies(s0, 0):
                cp.start()

        @pl.loop(0, ZITERS)
        def _(i):
            cid = i * NS + s

            @pl.when(cid < NZCH)
            def _():
                pltpu.sync_copy(zbuf, acc.at[pl.ds(cid * ZR, ZR)])

        plsc.subcore_barrier()

        @pl.loop(0, ssi)
        def _(it):
            ss = it * NW + wid
            p = lax.rem(it, 2)

            @pl.when(ss < nss)
            def _():
                for cp in idx_copies(ss, p):
                    cp.wait()
                ss2 = ss + NW

                @pl.when(ss2 < nss)
                def _():
                    for cp in idx_copies(ss2, 1 - p):
                        cp.start()

                for j in range(sbc):
                    gather_copy(p, j, 0, j).start()
                for sb in range(nsb):
                    cset = sb % 2
                    for j in range(sbc):
                        gather_copy(p, sb * sbc + j, cset, j).wait()
                    if sb < nsb - 1:
                        for j in range(sbc):
                            gather_copy(p, (sb + 1) * sbc + j,
                                        1 - cset, j).start()
                    for j in range(sbc):
                        pltpu.sync_copy(
                            rows.at[cset, pl.ds(j * CHUNK, CHUNK)],
                            acc.at[didx.at[p, sb * sbc + j]], add=True)

                if compact:
                    @pl.loop(0, ssc)
                    def _(ci):
                        for g in range(CHUNK // L):
                            dv = didx[p, ci, pl.ds(g * L, L)]
                            mask = dv < 128
                            npos = jnp.max(
                                plsc.all_reduce_population_count(mask))

                            @pl.when(npos > 0)
                            def _():
                                gc = cms[0]

                                @pl.when(gc < CAPG)
                                def _():
                                    sv = sidx[p, ci, pl.ds(g * L, L)]
                                    gr = gc // 8
                                    go = lax.rem(gc, 8) * L
                                    dump = 128 + jax.lax.iota(jnp.int32, 16)
                                    ldst[gr, pl.ds(go, L)] = (
                                        jnp.where(mask, dv, dump))
                                    lsrc[gr, pl.ds(go, L)] = (
                                        jnp.where(mask, sv, 0))
                                    cms[0] = gc + 1

        plsc.subcore_barrier()

        @pl.loop(0, ZITERS)
        def _(i):
            cid = i * NS + s

            @pl.when(cid < NZCH)
            def _():
                pltpu.sync_copy(acc.at[pl.ds(cid * ZR, ZR)],
                                out_hbm.at[c, pl.ds(cid * ZR, ZR)])

        if compact:
            pltpu.sync_copy(lsrc, ssrc_hbm.at[wid])
            pltpu.sync_copy(ldst, sdst_hbm.at[wid])
            cbuf[...] = jnp.zeros((L,), jnp.int32) + cms[0]
            pltpu.sync_copy(cbuf, cnt_hbm.at[wid])

    return pl.kernel(body, out_type=out_type, mesh=_mesh,
                     scratch_types=scratch, compiler_params=_sc_params)


_sc_agg16c = _make_sc_agg(16, True, 32, 4)
_sc_agg16 = _make_sc_agg(16, False, 32, 4)


def _make_sc_agg_colsplit(ssc, sbc):
    nss = NCHUNKP // ssc
    ssi = (nss + NS - 1) // NS
    nsb = ssc // sbc
    dp = 16

    def body(h_hbm, e_hbm, out_hbm,
             acc, sidx, didx, rows, zbuf, gsem, isem):
        c = lax.axis_index("c")
        s = lax.axis_index("s")

        def idx_copies(ss, p):
            base = ss * ssc
            return (
                pltpu.make_async_copy(e_hbm.at[0, pl.ds(base, ssc)],
                                      sidx.at[p], isem),
                pltpu.make_async_copy(e_hbm.at[1, pl.ds(base, ssc)],
                                      didx.at[p], isem),
            )

        def gather_copy(p, ci, rset, j):
            return pltpu.make_async_copy(
                h_hbm.at[c].at[sidx.at[p, ci]],
                rows.at[rset, pl.ds(j * CHUNK, CHUNK)], gsem)

        _zero_vmem_2d(zbuf, ZR, dp)

        @pl.when(s < nss)
        def _():
            for cp in idx_copies(s, 0):
                cp.start()

        @pl.loop(0, ZITERS)
        def _(i):
            cid = i * NS + s

            @pl.when(cid < NZCH)
            def _():
                pltpu.sync_copy(zbuf, acc.at[pl.ds(cid * ZR, ZR)])

        plsc.subcore_barrier()

        @pl.loop(0, ssi)
        def _(it):
            ss = it * NS + s
            p = lax.rem(it, 2)

            @pl.when(ss < nss)
            def _():
                for cp in idx_copies(ss, p):
                    cp.wait()
                ss2 = ss + NS

                @pl.when(ss2 < nss)
                def _():
                    for cp in idx_copies(ss2, 1 - p):
                        cp.start()

                for j in range(sbc):
                    gather_copy(p, j, 0, j).start()
                for sb in range(nsb):
                    cset = sb % 2
                    for j in range(sbc):
                        gather_copy(p, sb * sbc + j, cset, j).wait()
                    if sb < nsb - 1:
                        for j in range(sbc):
                            gather_copy(p, (sb + 1) * sbc + j,
                                        1 - cset, j).start()
                    for j in range(sbc):
                        pltpu.sync_copy(
                            rows.at[cset, pl.ds(j * CHUNK, CHUNK)],
                            acc.at[didx.at[p, sb * sbc + j]], add=True)

        plsc.subcore_barrier()

        @pl.loop(0, ZITERS)
        def _(i):
            cid = i * NS + s

            @pl.when(cid < NZCH)
            def _():
                pltpu.sync_copy(acc.at[pl.ds(cid * ZR, ZR)],
                                out_hbm.at[c, pl.ds(cid * ZR, ZR)])

    return pl.kernel(
        body,
        out_type=[jax.ShapeDtypeStruct((NC, N, dp), jnp.float32)],
        mesh=_mesh,
        scratch_types=[
            pltpu.VMEM_SHARED((NACC, dp), jnp.float32),
            pltpu.VMEM((2, ssc, CHUNK), jnp.int32),
            pltpu.VMEM((2, ssc, CHUNK), jnp.int32),
            pltpu.VMEM((2, sbc * CHUNK, dp), jnp.float32),
            pltpu.VMEM((ZR, dp), jnp.float32),
            pltpu.SemaphoreType.DMA,
            pltpu.SemaphoreType.DMA,
        ],
        compiler_params=_sc_params)


_sc_agg32cs = _make_sc_agg_colsplit(32, 4)


def _sc_layer4_body(h3_hbm, ssrc_hbm, sdst_hbm, cnt_hbm, out_hbm,
                    acc, lsrc, ldst, rows, zbuf, cbuf, sem):
    c = lax.axis_index("c")
    s = lax.axis_index("s")
    wid = s * NC + c

    _zero_vmem_2d(zbuf, ACC4 // NS, 64)
    pltpu.sync_copy(zbuf, acc.at[pl.ds(s * (ACC4 // NS), ACC4 // NS)])
    pltpu.sync_copy(ssrc_hbm.at[wid], lsrc)
    pltpu.sync_copy(sdst_hbm.at[wid], ldst)
    pltpu.sync_copy(cnt_hbm.at[wid], cbuf)
    cnt = jnp.max(cbuf[...])
    cnt8 = (cnt + 7) // 8 * 8
    dump_d = 128 + jax.lax.iota(jnp.int32, L)
    dump_s = jnp.zeros((L,), jnp.int32)

    @pl.loop(0, 8)
    def _(k):
        g = cnt + k

        @pl.when(g < cnt8)
        def _():
            gr = g // 8
            go = lax.rem(g, 8) * L
            ldst[gr, pl.ds(go, L)] = dump_d
            lsrc[gr, pl.ds(go, L)] = dump_s

    plsc.subcore_barrier()
    nblk = cnt8 // 8

    @pl.loop(0, CAPB)
    def _(g8):
        @pl.when(g8 < nblk)
        def _():
            pltpu.async_copy(h3_hbm.at[lsrc.at[g8]], rows, sem).wait()
            pltpu.sync_copy(rows, acc.at[ldst.at[g8]], add=True)

    plsc.subcore_barrier()

    @pl.when(s < 8)
    def _():
        pltpu.sync_copy(acc.at[pl.ds(s * 16, 16)],
                        out_hbm.at[c, pl.ds(s * 16, 16)])


_sc_layer4 = pl.kernel(
    _sc_layer4_body,
    out_type=[jax.ShapeDtypeStruct((NC, 128, 64), jnp.float32)],
    mesh=_mesh,
    scratch_types=[
        pltpu.VMEM_SHARED((ACC4, 64), jnp.float32),
        pltpu.VMEM((CAPB, CHUNK), jnp.int32),
        pltpu.VMEM((CAPB, CHUNK), jnp.int32),
        pltpu.VMEM((CHUNK, 64), jnp.float32),
        pltpu.VMEM((ACC4 // NS, 64), jnp.float32),
        pltpu.VMEM((L,), jnp.int32),
        pltpu.SemaphoreType.DMA,
    ],
    compiler_params=_sc_params)


def _gin_dense(hp, a0, a1, wt, br):
    din = hp.shape[1]
    dout = wt.shape[1]
    bn = 2000

    def body(h_ref, a0_ref, a1_ref, w_ref, b_ref, o_ref):
        x = h_ref[...] + a0_ref[...] + a1_ref[...]
        y = jnp.dot(x, w_ref[...], preferred_element_type=jnp.float32)
        o_ref[...] = jnp.maximum(y + b_ref[...], 0.0)

    return pl.pallas_call(
        body,
        grid=(N // bn,),
        in_specs=[
            pl.BlockSpec((bn, din), lambda i: (i, 0)),
            pl.BlockSpec((bn, din), lambda i: (i, 0)),
            pl.BlockSpec((bn, din), lambda i: (i, 0)),
            pl.BlockSpec((din, dout), lambda i: (0, 0)),
            pl.BlockSpec((1, dout), lambda i: (0, 0)),
        ],
        out_specs=pl.BlockSpec((bn, dout), lambda i: (i, 0)),
        out_shape=jax.ShapeDtypeStruct((N, dout), jnp.float32),
    )(hp, a0, a1, wt, br)


def _gin_dense_split(hp, a0, a1, wt, br):
    din = hp.shape[1]
    bn = 2000

    def body(h_ref, a0_ref, a1_ref, w_ref, b_ref, o_ref):
        x = h_ref[...] + a0_ref[...] + a1_ref[...]
        y = jnp.dot(x, w_ref[...], preferred_element_type=jnp.float32)
        y = jnp.maximum(y + b_ref[...], 0.0)
        o_ref[0] = y[:, 0:16]
        o_ref[1] = y[:, 16:32]

    return pl.pallas_call(
        body,
        grid=(N // bn,),
        in_specs=[
            pl.BlockSpec((bn, din), lambda i: (i, 0)),
            pl.BlockSpec((bn, din), lambda i: (i, 0)),
            pl.BlockSpec((bn, din), lambda i: (i, 0)),
            pl.BlockSpec((din, 32), lambda i: (0, 0)),
            pl.BlockSpec((1, 32), lambda i: (0, 0)),
        ],
        out_specs=pl.BlockSpec((2, bn, 16), lambda i: (0, i, 0)),
        out_shape=jax.ShapeDtypeStruct((2, N, 16), jnp.float32),
    )(hp, a0, a1, wt, br)


def _gin_dense_join(hs, ag, wt, br):
    dout = wt.shape[1]
    bn = 2000

    def body(h_ref, a_ref, w_ref, b_ref, o_ref):
        x = jnp.concatenate([h_ref[0] + a_ref[0],
                             h_ref[1] + a_ref[1]], axis=1)
        y = jnp.dot(x, w_ref[...], preferred_element_type=jnp.float32)
        o_ref[...] = jnp.maximum(y + b_ref[...], 0.0)

    return pl.pallas_call(
        body,
        grid=(N // bn,),
        in_specs=[
            pl.BlockSpec((2, bn, 16), lambda i: (0, i, 0)),
            pl.BlockSpec((2, bn, 16), lambda i: (0, i, 0)),
            pl.BlockSpec((32, dout), lambda i: (0, 0)),
            pl.BlockSpec((1, dout), lambda i: (0, 0)),
        ],
        out_specs=pl.BlockSpec((bn, dout), lambda i: (i, 0)),
        out_shape=jax.ShapeDtypeStruct((N, dout), jnp.float32),
    )(hs, ag, wt, br)


def _tail_body(h3s_ref, a40_ref, a41_ref, w4t_ref, b4_ref,
               wih1_ref, whh1_ref, bih1_ref, bhh1_ref,
               wih2_ref, whh2_ref, bih2_ref, bhh2_ref,
               wih3_ref, whh3_ref, bih3_ref, bhh3_ref,
               wgt_ref, bg_ref, wl1_ref, bl1_ref, wl2_ref, bl2_ref,
               wl3_ref, bl3_ref, o_ref):
    pre = h3s_ref[...] + a40_ref[...] + a41_ref[...]
    h4 = jnp.dot(pre, w4t_ref[...], preferred_element_type=jnp.float32)
    h4 = jnp.maximum(h4 + b4_ref[...], 0.0)
    feat = h4.reshape(16, 8, 128)

    def s2s(wih, whh, bihr, bhhr):
        qs = jnp.zeros((16, 256), jnp.float32)
        hh = jnp.zeros((16, 128), jnp.float32)
        cc = jnp.zeros((16, 128), jnp.float32)
        for _ in range(2):
            gates = (jnp.dot(qs, wih, preferred_element_type=jnp.float32)
                     + bihr
                     + jnp.dot(hh, whh, preferred_element_type=jnp.float32)
                     + bhhr)
            ig = lax.logistic(gates[:, 0:128])
            fg = lax.logistic(gates[:, 128:256])
            gg = jnp.tanh(gates[:, 256:384])
            og = lax.logistic(gates[:, 384:512])
            cc = fg * cc + ig * gg
            hh = og * jnp.tanh(cc)
            e = jnp.sum(feat * hh[:, None, :], axis=2)
            m = jnp.max(e, axis=1, keepdims=True)
            ex = jnp.exp(e - m)
            alpha = ex / jnp.sum(ex, axis=1, keepdims=True)
            r = jnp.sum(alpha[:, :, None] * feat, axis=1)
            qs = jnp.concatenate([hh, r], axis=1)
        return qs

    r1 = s2s(wih1_ref[...], whh1_ref[...], bih1_ref[...], bhh1_ref[...])
    r2 = s2s(wih2_ref[...], whh2_ref[...], bih2_ref[...], bhh2_ref[...])
    r3 = s2s(wih3_ref[...], whh3_ref[...], bih3_ref[...], bhh3_ref[...])
    readout = jnp.concatenate([r1, r2, r3], axis=1)
    t1 = jnp.dot(readout, wgt_ref[...],
                 preferred_element_type=jnp.float32) + bg_ref[0, 0]
    y1 = jnp.tanh(jnp.dot(wl1_ref[...], t1,
                          preferred_element_type=jnp.float32) + bl1_ref[...])
    y2 = jnp.maximum(jnp.dot(wl2_ref[...], y1,
                             preferred_element_type=jnp.float32)
                     + bl2_ref[...], 0.0)
    y3 = jnp.dot(wl3_ref[...], y2,
                 preferred_element_type=jnp.float32) + bl3_ref[...]
    o_ref[...] = lax.logistic(y3)


def _tail(h3s, a40, a41, args):
    return pl.pallas_call(
        _tail_body,
        out_shape=jax.ShapeDtypeStruct((1, 1), jnp.float32),
    )(h3s, a40, a41, *args)


def kernel(h, edge_index, pathway_nodes, W1, b1, W2, b2, W3, b3, W4, b4,
           Wih1, Whh1, bih1, bhh1, Wih2, Whh2, bih2, bhh2,
           Wih3, Whh3, bih3, bhh3, Wg, bg, Wl1, bl1, Wl2, bl2, Wl3, bl3):
    f32 = jnp.float32
    hp = jnp.pad(h, ((0, 0), (0, 16 - h.shape[1])))
    w1t = jnp.pad(W1.T, ((0, 12), (0, 4)))
    b1r = jnp.pad(b1, (0, 4)).reshape(1, 16)
    w2t = jnp.pad(W2.T, ((0, 4), (0, 0)))
    b2r = b2.reshape(1, 32)
    w3t = W3.T
    b3r = b3.reshape(1, 64)

    e3 = edge_index.reshape(2, NCHUNK, CHUNK)
    npad = NCHUNKP - NCHUNK
    pad = jnp.stack([jnp.zeros((npad, CHUNK), jnp.int32),
                     jnp.full((npad, CHUNK), N, jnp.int32)])
    e3 = jnp.concatenate([e3, pad], axis=1)

    agg1, ssrc, sdst, cnts = _sc_agg16c(hp, e3)
    h1 = _gin_dense(hp, agg1[0], agg1[1], w1t, b1r)
    (agg2,) = _sc_agg16(h1, e3)
    h2s = _gin_dense_split(h1, agg2[0], agg2[1], w2t, b2r)
    (agg3,) = _sc_agg32cs(h2s, e3)
    h3 = _gin_dense_join(h2s, agg3, w3t, b3r)
    (agg4,) = _sc_layer4(h3, ssrc, sdst, cnts)

    tail_args = (
        W4.T.astype(f32), b4.reshape(1, 128),
        Wih1.T, Whh1.T, bih1.reshape(1, 512), bhh1.reshape(1, 512),
        Wih2.T, Whh2.T, bih2.reshape(1, 512), bhh2.reshape(1, 512),
        Wih3.T, Whh3.T, bih3.reshape(1, 512), bhh3.reshape(1, 512),
        Wg.T, bg.reshape(1, 1),
        Wl1, bl1.reshape(256, 1), Wl2, bl2.reshape(64, 1),
        Wl3, bl3.reshape(1, 1),
    )
    return _tail(h3[0:128], agg4[0], agg4[1], tail_args)


# speedup vs baseline: 1.9144x; 1.9144x over previous
import dataclasses

import jax
import jax.numpy as jnp
from jax import lax
from jax.experimental import pallas as pl
from jax.experimental.pallas import tpu as pltpu
from jax.experimental.pallas import tpu_sc as plsc

N = 50000
E = 1600000
NC = 2
NS = 16
NW = NC * NS
L = 16

CHUNK = 128
NCHUNK = E // CHUNK
SSC = 32
NCHUNKP = ((NCHUNK + SSC - 1) // SSC) * SSC
NSS = NCHUNKP // SSC
SSI = (NSS + NW - 1) // NW
NACC = N + 16
ZR = 80
NZCH = N // ZR
ZITERS = (NZCH + NS - 1) // NS
CAPG = 512
CAPB = CAPG // 8
ACC4 = 144

_mesh = plsc.VectorSubcoreMesh(core_axis_name="c", subcore_axis_name="s",
                               num_cores=NC, num_subcores=NS)

_sc_params = pltpu.CompilerParams(needs_layout_passes=False,
                                  use_tc_tiling_on_sc=False)


def _zero_vmem_2d(ref, rows, dp):
    z = jnp.zeros((L,), jnp.float32)

    @pl.loop(0, rows)
    def _(r):
        for j in range(0, dp, L):
            ref[r, pl.ds(j, L)] = z


def _make_sc_agg(dp, compact, ssc, sbc):
    nss = NCHUNKP // ssc
    ssi = (nss + NW - 1) // NW
    nsb = ssc // sbc
    out_type = [jax.ShapeDtypeStruct((NC, N, dp), jnp.float32)]
    if compact:
        out_type += [
            jax.ShapeDtypeStruct((NW, CAPB, CHUNK), jnp.int32),
            jax.ShapeDtypeStruct((NW, CAPB, CHUNK), jnp.int32),
            jax.ShapeDtypeStruct((NW, L), jnp.int32),
        ]
    scratch = [
        pltpu.VMEM_SHARED((NACC, dp), jnp.float32),
        pltpu.VMEM((2, ssc, CHUNK), jnp.int32),
        pltpu.VMEM((2, ssc, CHUNK), jnp.int32),
        pltpu.VMEM((2, sbc * CHUNK, dp), jnp.float32),
        pltpu.VMEM((ZR, dp), jnp.float32),
        pltpu.SemaphoreType.DMA,
        pltpu.SemaphoreType.DMA,
    ]
    if compact:
        scratch += [
            pltpu.VMEM((CAPB, CHUNK), jnp.int32),
            pltpu.VMEM((CAPB, CHUNK), jnp.int32),
            pltpu.VMEM((L,), jnp.int32),
            pltpu.SMEM((8,), jnp.int32),
        ]

    def body(h_hbm, e_hbm, *rest):
        if compact:
            (out_hbm, ssrc_hbm, sdst_hbm, cnt_hbm,
             acc, sidx, didx, rows, zbuf, gsem, isem,
             lsrc, ldst, cbuf, cms) = rest
        else:
            out_hbm, acc, sidx, didx, rows, zbuf, gsem, isem = rest
        c = lax.axis_index("c")
        s = lax.axis_index("s")
        wid = s * NC + c

        def idx_copies(ss, p):
            base = ss * ssc
            return (
                pltpu.make_async_copy(e_hbm.at[0, pl.ds(base, ssc)],
                                      sidx.at[p], isem),
                pltpu.make_async_copy(e_hbm.at[1, pl.ds(base, ssc)],
                                      didx.at[p], isem),
            )

        def gather_copy(p, ci, rset, j):
            return pltpu.make_async_copy(
                h_hbm.at[sidx.at[p, ci]],
                rows.at[rset, pl.ds(j * CHUNK, CHUNK)], gsem)

        _zero_vmem_2d(zbuf, ZR, dp)
        if compact:
            cms[0] = 0

        s0 = wid

        @pl.when(s0 < nss)
        def _():
            for cp in idx_copies(s0, 0):
                cp.start()

        @pl.loop(0, ZITERS)
        def _(i):
            cid = i * NS + s

            @pl.when(cid < NZCH)
            def _():
                pltpu.sync_copy(zbuf, acc.at[pl.ds(cid * ZR, ZR)])

        plsc.subcore_barrier()

        @pl.loop(0, ssi)
        def _(it):
            ss = it * NW + wid
            p = lax.rem(it, 2)

            @pl.when(ss < nss)
            def _():
                for cp in idx_copies(ss, p):
                    cp.wait()
                ss2 = ss + NW

                @pl.when(ss2 < nss)
                def _():
                    for cp in idx_copies(ss2, 1 - p):
                        cp.start()

                for j in range(sbc):
                    gather_copy(p, j, 0, j).start()
                for sb in range(nsb):
                    cset = sb % 2
                    for j in range(sbc):
                        gather_copy(p, sb * sbc + j, cset, j).wait()
                    if sb < nsb - 1:
                        for j in range(sbc):
                            gather_copy(p, (sb + 1) * sbc + j,
                                        1 - cset, j).start()
                    for j in range(sbc):
                        pltpu.sync_copy(
                            rows.at[cset, pl.ds(j * CHUNK, CHUNK)],
                            acc.at[didx.at[p, sb * sbc + j]], add=True)

                if compact:
                    @pl.loop(0, ssc)
                    def _(ci):
                        for g in range(CHUNK // L):
                            dv = didx[p, ci, pl.ds(g * L, L)]
                            mask = dv < 128
                            npos = jnp.max(
                                plsc.all_reduce_population_count(mask))

                            @pl.when(npos > 0)
                            def _():
                                gc = cms[0]

                                @pl.when(gc < CAPG)
                                def _():
                                    sv = sidx[p, ci, pl.ds(g * L, L)]
                                    gr = gc // 8
                                    go = lax.rem(gc, 8) * L
                                    iot = jax.lax.iota(jnp.int32, 16)
                                    ldst[gr, pl.ds(go, L)] = (
                                        jnp.where(mask, dv, 128 + iot))
                                    lsrc[gr, pl.ds(go, L)] = (
                                        jnp.where(mask, sv, gc * L + iot))
                                    cms[0] = gc + 1

        plsc.subcore_barrier()

        @pl.loop(0, ZITERS)
        def _(i):
            cid = i * NS + s

            @pl.when(cid < NZCH)
            def _():
                pltpu.sync_copy(acc.at[pl.ds(cid * ZR, ZR)],
                                out_hbm.at[c, pl.ds(cid * ZR, ZR)])

        if compact:
            pltpu.sync_copy(lsrc, ssrc_hbm.at[wid])
            pltpu.sync_copy(ldst, sdst_hbm.at[wid])
            cbuf[...] = jnp.zeros((L,), jnp.int32) + cms[0]
            pltpu.sync_copy(cbuf, cnt_hbm.at[wid])

    return pl.kernel(body, out_type=out_type, mesh=_mesh,
                     scratch_types=scratch, compiler_params=_sc_params)


_sc_agg16c = _make_sc_agg(16, True, 32, 4)
_sc_agg16 = _make_sc_agg(16, False, 32, 4)


def _make_sc_agg_colsplit(ssc, sbc):
    nss = NCHUNKP // ssc
    ssi = (nss + NS - 1) // NS
    nsb = ssc // sbc
    dp = 16

    def body(h_hbm, e_hbm, out_hbm,
             acc, sidx, didx, rows, zbuf, gsem, isem):
        c = lax.axis_index("c")
        s = lax.axis_index("s")

        def idx_copies(ss, p):
            base = ss * ssc
            return (
                pltpu.make_async_copy(e_hbm.at[0, pl.ds(base, ssc)],
                                      sidx.at[p], isem),
                pltpu.make_async_copy(e_hbm.at[1, pl.ds(base, ssc)],
                                      didx.at[p], isem),
            )

        def gather_copy(p, ci, rset, j):
            return pltpu.make_async_copy(
                h_hbm.at[c].at[sidx.at[p, ci]],
                rows.at[rset, pl.ds(j * CHUNK, CHUNK)], gsem)

        _zero_vmem_2d(zbuf, ZR, dp)

        @pl.when(s < nss)
        def _():
            for cp in idx_copies(s, 0):
                cp.start()

        @pl.loop(0, ZITERS)
        def _(i):
            cid = i * NS + s

            @pl.when(cid < NZCH)
            def _():
                pltpu.sync_copy(zbuf, acc.at[pl.ds(cid * ZR, ZR)])

        plsc.subcore_barrier()

        @pl.loop(0, ssi)
        def _(it):
            ss = it * NS + s
            p = lax.rem(it, 2)

            @pl.when(ss < nss)
            def _():
                for cp in idx_copies(ss, p):
                    cp.wait()
                ss2 = ss + NS

                @pl.when(ss2 < nss)
                def _():
                    for cp in idx_copies(ss2, 1 - p):
                        cp.start()

                for j in range(sbc):
                    gather_copy(p, j, 0, j).start()
                for sb in range(nsb):
                    cset = sb % 2
                    for j in range(sbc):
                        gather_copy(p, sb * sbc + j, cset, j).wait()
                    if sb < nsb - 1:
                        for j in range(sbc):
                            gather_copy(p, (sb + 1) * sbc + j,
                                        1 - cset, j).start()
                    for j in range(sbc):
                        pltpu.sync_copy(
                            rows.at[cset, pl.ds(j * CHUNK, CHUNK)],
                            acc.at[didx.at[p, sb * sbc + j]], add=True)

        plsc.subcore_barrier()

        @pl.loop(0, ZITERS)
        def _(i):
            cid = i * NS + s

            @pl.when(cid < NZCH)
            def _():
                pltpu.sync_copy(acc.at[pl.ds(cid * ZR, ZR)],
                                out_hbm.at[c, pl.ds(cid * ZR, ZR)])

    return pl.kernel(
        body,
        out_type=[jax.ShapeDtypeStruct((NC, N, dp), jnp.float32)],
        mesh=_mesh,
        scratch_types=[
            pltpu.VMEM_SHARED((NACC, dp), jnp.float32),
            pltpu.VMEM((2, ssc, CHUNK), jnp.int32),
            pltpu.VMEM((2, ssc, CHUNK), jnp.int32),
            pltpu.VMEM((2, sbc * CHUNK, dp), jnp.float32),
            pltpu.VMEM((ZR, dp), jnp.float32),
            pltpu.SemaphoreType.DMA,
            pltpu.SemaphoreType.DMA,
        ],
        compiler_params=_sc_params)


_sc_agg32cs = _make_sc_agg_colsplit(32, 4)


def _sc_layer4_body(h3_hbm, ssrc_hbm, sdst_hbm, cnt_hbm, out_hbm,
                    acc, lsrc, ldst, rows, zbuf, cbuf, sem):
    c = lax.axis_index("c")
    s = lax.axis_index("s")
    wid = s * NC + c

    _zero_vmem_2d(zbuf, ACC4 // NS, 64)
    pltpu.sync_copy(zbuf, acc.at[pl.ds(s * (ACC4 // NS), ACC4 // NS)])
    pltpu.sync_copy(ssrc_hbm.at[wid], lsrc)
    pltpu.sync_copy(sdst_hbm.at[wid], ldst)
    pltpu.sync_copy(cnt_hbm.at[wid], cbuf)
    cnt = jnp.max(cbuf[...])
    cnt8 = (cnt + 7) // 8 * 8
    dump_d = 128 + jax.lax.iota(jnp.int32, L)

    @pl.loop(0, 8)
    def _(k):
        g = cnt + k

        @pl.when(g < cnt8)
        def _():
            gr = g // 8
            go = lax.rem(g, 8) * L
            ldst[gr, pl.ds(go, L)] = dump_d
            lsrc[gr, pl.ds(go, L)] = g * L + jax.lax.iota(jnp.int32, L)

    plsc.subcore_barrier()
    nblk = cnt8 // 8

    @pl.loop(0, CAPB)
    def _(g8):
        @pl.when(g8 < nblk)
        def _():
            pltpu.async_copy(h3_hbm.at[lsrc.at[g8]], rows, sem).wait()
            pltpu.sync_copy(rows, acc.at[ldst.at[g8]], add=True)

    plsc.subcore_barrier()

    @pl.when(s < 8)
    def _():
        pltpu.sync_copy(acc.at[pl.ds(s * 16, 16)],
                        out_hbm.at[c, pl.ds(s * 16, 16)])


_sc_layer4 = pl.kernel(
    _sc_layer4_body,
    out_type=[jax.ShapeDtypeStruct((NC, 128, 64), jnp.float32)],
    mesh=_mesh,
    scratch_types=[
        pltpu.VMEM_SHARED((ACC4, 64), jnp.float32),
        pltpu.VMEM((CAPB, CHUNK), jnp.int32),
        pltpu.VMEM((CAPB, CHUNK), jnp.int32),
        pltpu.VMEM((CHUNK, 64), jnp.float32),
        pltpu.VMEM((ACC4 // NS, 64), jnp.float32),
        pltpu.VMEM((L,), jnp.int32),
        pltpu.SemaphoreType.DMA,
    ],
    compiler_params=_sc_params)


def _gin_dense(hp, a0, a1, wt, br):
    din = hp.shape[1]
    dout = wt.shape[1]
    bn = 2000

    def body(h_ref, a0_ref, a1_ref, w_ref, b_ref, o_ref):
        x = h_ref[...] + a0_ref[...] + a1_ref[...]
        y = jnp.dot(x, w_ref[...], preferred_element_type=jnp.float32)
        o_ref[...] = jnp.maximum(y + b_ref[...], 0.0)

    return pl.pallas_call(
        body,
        grid=(N // bn,),
        in_specs=[
            pl.BlockSpec((bn, din), lambda i: (i, 0)),
            pl.BlockSpec((bn, din), lambda i: (i, 0)),
            pl.BlockSpec((bn, din), lambda i: (i, 0)),
            pl.BlockSpec((din, dout), lambda i: (0, 0)),
            pl.BlockSpec((1, dout), lambda i: (0, 0)),
        ],
        out_specs=pl.BlockSpec((bn, dout), lambda i: (i, 0)),
        out_shape=jax.ShapeDtypeStruct((N, dout), jnp.float32),
    )(hp, a0, a1, wt, br)


def _gin_dense_split(hp, a0, a1, wt, br):
    din = hp.shape[1]
    bn = 2000

    def body(h_ref, a0_ref, a1_ref, w_ref, b_ref, o_ref):
        x = h_ref[...] + a0_ref[...] + a1_ref[...]
        y = jnp.dot(x, w_ref[...], preferred_element_type=jnp.float32)
        y = jnp.maximum(y + b_ref[...], 0.0)
        o_ref[0] = y[:, 0:16]
        o_ref[1] = y[:, 16:32]

    return pl.pallas_call(
        body,
        grid=(N // bn,),
        in_specs=[
            pl.BlockSpec((bn, din), lambda i: (i, 0)),
            pl.BlockSpec((bn, din), lambda i: (i, 0)),
            pl.BlockSpec((bn, din), lambda i: (i, 0)),
            pl.BlockSpec((din, 32), lambda i: (0, 0)),
            pl.BlockSpec((1, 32), lambda i: (0, 0)),
        ],
        out_specs=pl.BlockSpec((2, bn, 16), lambda i: (0, i, 0)),
        out_shape=jax.ShapeDtypeStruct((2, N, 16), jnp.float32),
    )(hp, a0, a1, wt, br)


def _gin_dense_join(hs, ag, wt, br):
    dout = wt.shape[1]
    bn = 2000

    def body(h_ref, a_ref, w_ref, b_ref, o_ref):
        x = jnp.concatenate([h_ref[0] + a_ref[0],
                             h_ref[1] + a_ref[1]], axis=1)
        y = jnp.dot(x, w_ref[...], preferred_element_type=jnp.float32)
        o_ref[...] = jnp.maximum(y + b_ref[...], 0.0)

    return pl.pallas_call(
        body,
        grid=(N // bn,),
        in_specs=[
            pl.BlockSpec((2, bn, 16), lambda i: (0, i, 0)),
            pl.BlockSpec((2, bn, 16), lambda i: (0, i, 0)),
            pl.BlockSpec((32, dout), lambda i: (0, 0)),
            pl.BlockSpec((1, dout), lambda i: (0, 0)),
        ],
        out_specs=pl.BlockSpec((bn, dout), lambda i: (i, 0)),
        out_shape=jax.ShapeDtypeStruct((N, dout), jnp.float32),
    )(hs, ag, wt, br)


def _tail_body(h3s_ref, a40_ref, a41_ref, w4t_ref, b4_ref,
               wih1_ref, whh1_ref, bih1_ref, bhh1_ref,
               wih2_ref, whh2_ref, bih2_ref, bhh2_ref,
               wih3_ref, whh3_ref, bih3_ref, bhh3_ref,
               wgt_ref, bg_ref, wl1_ref, bl1_ref, wl2_ref, bl2_ref,
               wl3_ref, bl3_ref, o_ref):
    pre = h3s_ref[...] + a40_ref[...] + a41_ref[...]
    h4 = jnp.dot(pre, w4t_ref[...], preferred_element_type=jnp.float32)
    h4 = jnp.maximum(h4 + b4_ref[...], 0.0)
    feat = h4.reshape(16, 8, 128)

    def s2s(wih, whh, bihr, bhhr):
        qs = jnp.zeros((16, 256), jnp.float32)
        hh = jnp.zeros((16, 128), jnp.float32)
        cc = jnp.zeros((16, 128), jnp.float32)
        for _ in range(2):
            gates = (jnp.dot(qs, wih, preferred_element_type=jnp.float32)
                     + bihr
                     + jnp.dot(hh, whh, preferred_element_type=jnp.float32)
                     + bhhr)
            ig = lax.logistic(gates[:, 0:128])
            fg = lax.logistic(gates[:, 128:256])
            gg = jnp.tanh(gates[:, 256:384])
            og = lax.logistic(gates[:, 384:512])
            cc = fg * cc + ig * gg
            hh = og * jnp.tanh(cc)
            e = jnp.sum(feat * hh[:, None, :], axis=2)
            m = jnp.max(e, axis=1, keepdims=True)
            ex = jnp.exp(e - m)
            alpha = ex / jnp.sum(ex, axis=1, keepdims=True)
            r = jnp.sum(alpha[:, :, None] * feat, axis=1)
            qs = jnp.concatenate([hh, r], axis=1)
        return qs

    r1 = s2s(wih1_ref[...], whh1_ref[...], bih1_ref[...], bhh1_ref[...])
    r2 = s2s(wih2_ref[...], whh2_ref[...], bih2_ref[...], bhh2_ref[...])
    r3 = s2s(wih3_ref[...], whh3_ref[...], bih3_ref[...], bhh3_ref[...])
    readout = jnp.concatenate([r1, r2, r3], axis=1)
    t1 = jnp.dot(readout, wgt_ref[...],
                 preferred_element_type=jnp.float32) + bg_ref[0, 0]
    y1 = jnp.tanh(jnp.dot(wl1_ref[...], t1,
                          preferred_element_type=jnp.float32) + bl1_ref[...])
    y2 = jnp.maximum(jnp.dot(wl2_ref[...], y1,
                             preferred_element_type=jnp.float32)
                     + bl2_ref[...], 0.0)
    y3 = jnp.dot(wl3_ref[...], y2,
                 preferred_element_type=jnp.float32) + bl3_ref[...]
    o_ref[...] = lax.logistic(y3)


def _tail(h3s, a40, a41, args):
    return pl.pallas_call(
        _tail_body,
        out_shape=jax.ShapeDtypeStruct((1, 1), jnp.float32),
    )(h3s, a40, a41, *args)


def kernel(h, edge_index, pathway_nodes, W1, b1, W2, b2, W3, b3, W4, b4,
           Wih1, Whh1, bih1, bhh1, Wih2, Whh2, bih2, bhh2,
           Wih3, Whh3, bih3, bhh3, Wg, bg, Wl1, bl1, Wl2, bl2, Wl3, bl3):
    f32 = jnp.float32
    hp = jnp.pad(h, ((0, 0), (0, 16 - h.shape[1])))
    w1t = jnp.pad(W1.T, ((0, 12), (0, 4)))
    b1r = jnp.pad(b1, (0, 4)).reshape(1, 16)
    w2t = jnp.pad(W2.T, ((0, 4), (0, 0)))
    b2r = b2.reshape(1, 32)
    w3t = W3.T
    b3r = b3.reshape(1, 64)

    e3 = edge_index.reshape(2, NCHUNK, CHUNK)
    npad = NCHUNKP - NCHUNK
    pad = jnp.stack([jnp.zeros((npad, CHUNK), jnp.int32),
                     jnp.full((npad, CHUNK), N, jnp.int32)])
    e3 = jnp.concatenate([e3, pad], axis=1)

    agg1, ssrc, sdst, cnts = _sc_agg16c(hp, e3)
    h1 = _gin_dense(hp, agg1[0], agg1[1], w1t, b1r)
    (agg2,) = _sc_agg16(h1, e3)
    h2s = _gin_dense_split(h1, agg2[0], agg2[1], w2t, b2r)
    (agg3,) = _sc_agg32cs(h2s, e3)
    h3 = _gin_dense_join(h2s, agg3, w3t, b3r)
    (agg4,) = _sc_layer4(h3, ssrc, sdst, cnts)

    tail_args = (
        W4.T.astype(f32), b4.reshape(1, 128),
        Wih1.T, Whh1.T, bih1.reshape(1, 512), bhh1.reshape(1, 512),
        Wih2.T, Whh2.T, bih2.reshape(1, 512), bhh2.reshape(1, 512),
        Wih3.T, Whh3.T, bih3.reshape(1, 512), bhh3.reshape(1, 512),
        Wg.T, bg.reshape(1, 1),
        Wl1, bl1.reshape(256, 1), Wl2, bl2.reshape(64, 1),
        Wl3, bl3.reshape(1, 1),
    )
    return _tail(h3[0:128], agg4[0], agg4[1], tail_args)


# speedup vs baseline: 2.1802x; 1.1388x over previous
import dataclasses

import jax
import jax.numpy as jnp
from jax import lax
from jax.experimental import pallas as pl
from jax.experimental.pallas import tpu as pltpu
from jax.experimental.pallas import tpu_sc as plsc

N = 50000
E = 1600000
NC = 2
NS = 16
NW = NC * NS
L = 16

CHUNK = 128
NCHUNK = E // CHUNK
SSC = 32
NCHUNKP = ((NCHUNK + SSC - 1) // SSC) * SSC
NSS = NCHUNKP // SSC
SSI = (NSS + NW - 1) // NW
NACC = N + 16
ZR = 400
NZCH = N // ZR
ZITERS = (NZCH + NS - 1) // NS
CAPG = 512
CAPB = CAPG // 8
ACC4 = 144

_mesh = plsc.VectorSubcoreMesh(core_axis_name="c", subcore_axis_name="s",
                               num_cores=NC, num_subcores=NS)

_sc_params = pltpu.CompilerParams(needs_layout_passes=False,
                                  use_tc_tiling_on_sc=False)


def _zero_vmem_2d(ref, rows, dp):
    z = jnp.zeros((L,), jnp.float32)

    @pl.loop(0, rows)
    def _(r):
        for j in range(0, dp, L):
            ref[r, pl.ds(j, L)] = z


def _make_sc_agg(dp, compact, ssc, sbc):
    nss = NCHUNKP // ssc
    ssi = (nss + NW - 1) // NW
    nsb = ssc // sbc
    out_type = [jax.ShapeDtypeStruct((NC, N, dp), jnp.float32)]
    if compact:
        out_type += [
            jax.ShapeDtypeStruct((NW, CAPB, CHUNK), jnp.int32),
            jax.ShapeDtypeStruct((NW, CAPB, CHUNK), jnp.int32),
            jax.ShapeDtypeStruct((NW, L), jnp.int32),
        ]
    scratch = [
        pltpu.VMEM_SHARED((NACC, dp), jnp.float32),
        pltpu.VMEM((2, ssc, CHUNK), jnp.int32),
        pltpu.VMEM((2, ssc, CHUNK), jnp.int32),
        pltpu.VMEM((2, sbc * CHUNK, dp), jnp.float32),
        pltpu.VMEM((ZR, dp), jnp.float32),
        pltpu.SemaphoreType.DMA,
        pltpu.SemaphoreType.DMA,
    ]
    if compact:
        scratch += [
            pltpu.VMEM((CAPB, CHUNK), jnp.int32),
            pltpu.VMEM((CAPB, CHUNK), jnp.int32),
            pltpu.VMEM((L,), jnp.int32),
            pltpu.SMEM((8,), jnp.int32),
        ]

    def body(h_hbm, e_hbm, *rest):
        if compact:
            (out_hbm, ssrc_hbm, sdst_hbm, cnt_hbm,
             acc, sidx, didx, rows, zbuf, gsem, isem,
             lsrc, ldst, cbuf, cms) = rest
        else:
            out_hbm, acc, sidx, didx, rows, zbuf, gsem, isem = rest
        c = lax.axis_index("c")
        s = lax.axis_index("s")
        wid = s * NC + c

        def idx_copies(ss, p):
            base = ss * ssc
            return (
                pltpu.make_async_copy(e_hbm.at[0, pl.ds(base, ssc)],
                                      sidx.at[p], isem),
                pltpu.make_async_copy(e_hbm.at[1, pl.ds(base, ssc)],
                                      didx.at[p], isem),
            )

        def gather_copy(p, ci, rset, j):
            return pltpu.make_async_copy(
                h_hbm.at[sidx.at[p, ci]],
                rows.at[rset, pl.ds(j * CHUNK, CHUNK)], gsem)

        _zero_vmem_2d(zbuf, ZR, dp)
        if compact:
            cms[0] = 0

        s0 = wid

        @pl.when(s0 < nss)
        def _():
            for cp in idx_copies(s0, 0):
                cp.start()

        @pl.loop(0, ZITERS)
        def _(i):
            cid = i * NS + s

            @pl.when(cid < NZCH)
            def _():
                pltpu.sync_copy(zbuf, acc.at[pl.ds(cid * ZR, ZR)])

        plsc.subcore_barrier()

        @pl.loop(0, ssi)
        def _(it):
            ss = it * NW + wid
            p = lax.rem(it, 2)

            @pl.when(ss < nss)
            def _():
                for cp in idx_copies(ss, p):
                    cp.wait()
                ss2 = ss + NW

                @pl.when(ss2 < nss)
                def _():
                    for cp in idx_copies(ss2, 1 - p):
                        cp.start()

                for j in range(sbc):
                    gather_copy(p, j, 0, j).start()
                for sb in range(nsb):
                    cset = sb % 2
                    for j in range(sbc):
                        gather_copy(p, sb * sbc + j, cset, j).wait()
                    if sb < nsb - 1:
                        for j in range(sbc):
                            gather_copy(p, (sb + 1) * sbc + j,
                                        1 - cset, j).start()
                    for j in range(sbc):
                        pltpu.sync_copy(
                            rows.at[cset, pl.ds(j * CHUNK, CHUNK)],
                            acc.at[didx.at[p, sb * sbc + j]], add=True)

                if compact:
                    @pl.loop(0, ssc)
                    def _(ci):
                        for g in range(CHUNK // L):
                            dv = didx[p, ci, pl.ds(g * L, L)]
                            mask = dv < 128
                            npos = jnp.max(
                                plsc.all_reduce_population_count(mask))

                            @pl.when(npos > 0)
                            def _():
                                gc = cms[0]

                                @pl.when(gc < CAPG)
                                def _():
                                    sv = sidx[p, ci, pl.ds(g * L, L)]
                                    gr = gc // 8
                                    go = lax.rem(gc, 8) * L
                                    iot = jax.lax.iota(jnp.int32, 16)
                                    ldst[gr, pl.ds(go, L)] = (
                                        jnp.where(mask, dv, 128 + iot))
                                    lsrc[gr, pl.ds(go, L)] = (
                                        jnp.where(mask, sv, gc * L + iot))
                                    cms[0] = gc + 1

        plsc.subcore_barrier()

        @pl.loop(0, ZITERS)
        def _(i):
            cid = i * NS + s

            @pl.when(cid < NZCH)
            def _():
                pltpu.sync_copy(acc.at[pl.ds(cid * ZR, ZR)],
                                out_hbm.at[c, pl.ds(cid * ZR, ZR)])

        if compact:
            pltpu.sync_copy(lsrc, ssrc_hbm.at[wid])
            pltpu.sync_copy(ldst, sdst_hbm.at[wid])
            cbuf[...] = jnp.zeros((L,), jnp.int32) + cms[0]
            pltpu.sync_copy(cbuf, cnt_hbm.at[wid])

    return pl.kernel(body, out_type=out_type, mesh=_mesh,
                     scratch_types=scratch, compiler_params=_sc_params)


_sc_agg16c = _make_sc_agg(16, True, 32, 8)
_sc_agg16 = _make_sc_agg(16, False, 32, 8)


def _make_sc_agg_colsplit(ssc, sbc):
    nss = NCHUNKP // ssc
    ssi = (nss + NS - 1) // NS
    nsb = ssc // sbc
    dp = 16

    def body(h_hbm, e_hbm, out_hbm,
             acc, sidx, didx, rows, zbuf, gsem, isem):
        c = lax.axis_index("c")
        s = lax.axis_index("s")

        def idx_copies(ss, p):
            base = ss * ssc
            return (
                pltpu.make_async_copy(e_hbm.at[0, pl.ds(base, ssc)],
                                      sidx.at[p], isem),
                pltpu.make_async_copy(e_hbm.at[1, pl.ds(base, ssc)],
                                      didx.at[p], isem),
            )

        def gather_copy(p, ci, rset, j):
            return pltpu.make_async_copy(
                h_hbm.at[c].at[sidx.at[p, ci]],
                rows.at[rset, pl.ds(j * CHUNK, CHUNK)], gsem)

        _zero_vmem_2d(zbuf, ZR, dp)

        @pl.when(s < nss)
        def _():
            for cp in idx_copies(s, 0):
                cp.start()

        @pl.loop(0, ZITERS)
        def _(i):
            cid = i * NS + s

            @pl.when(cid < NZCH)
            def _():
                pltpu.sync_copy(zbuf, acc.at[pl.ds(cid * ZR, ZR)])

        plsc.subcore_barrier()

        @pl.loop(0, ssi)
        def _(it):
            ss = it * NS + s
            p = lax.rem(it, 2)

            @pl.when(ss < nss)
            def _():
                for cp in idx_copies(ss, p):
                    cp.wait()
                ss2 = ss + NS

                @pl.when(ss2 < nss)
                def _():
                    for cp in idx_copies(ss2, 1 - p):
                        cp.start()

                for j in range(sbc):
                    gather_copy(p, j, 0, j).start()
                for sb in range(nsb):
                    cset = sb % 2
                    for j in range(sbc):
                        gather_copy(p, sb * sbc + j, cset, j).wait()
                    if sb < nsb - 1:
                        for j in range(sbc):
                            gather_copy(p, (sb + 1) * sbc + j,
                                        1 - cset, j).start()
                    for j in range(sbc):
                        pltpu.sync_copy(
                            rows.at[cset, pl.ds(j * CHUNK, CHUNK)],
                            acc.at[didx.at[p, sb * sbc + j]], add=True)

        plsc.subcore_barrier()

        @pl.loop(0, ZITERS)
        def _(i):
            cid = i * NS + s

            @pl.when(cid < NZCH)
            def _():
                pltpu.sync_copy(acc.at[pl.ds(cid * ZR, ZR)],
                                out_hbm.at[c, pl.ds(cid * ZR, ZR)])

    return pl.kernel(
        body,
        out_type=[jax.ShapeDtypeStruct((NC, N, dp), jnp.float32)],
        mesh=_mesh,
        scratch_types=[
            pltpu.VMEM_SHARED((NACC, dp), jnp.float32),
            pltpu.VMEM((2, ssc, CHUNK), jnp.int32),
            pltpu.VMEM((2, ssc, CHUNK), jnp.int32),
            pltpu.VMEM((2, sbc * CHUNK, dp), jnp.float32),
            pltpu.VMEM((ZR, dp), jnp.float32),
            pltpu.SemaphoreType.DMA,
            pltpu.SemaphoreType.DMA,
        ],
        compiler_params=_sc_params)


_sc_agg32cs = _make_sc_agg_colsplit(32, 8)


def _sc_layer4_body(h3_hbm, ssrc_hbm, sdst_hbm, cnt_hbm, out_hbm,
                    acc, lsrc, ldst, rows, zbuf, cbuf, sem):
    c = lax.axis_index("c")
    s = lax.axis_index("s")
    wid = s * NC + c

    _zero_vmem_2d(zbuf, ACC4 // NS, 64)
    pltpu.sync_copy(zbuf, acc.at[pl.ds(s * (ACC4 // NS), ACC4 // NS)])
    pltpu.sync_copy(ssrc_hbm.at[wid], lsrc)
    pltpu.sync_copy(sdst_hbm.at[wid], ldst)
    pltpu.sync_copy(cnt_hbm.at[wid], cbuf)
    cnt = jnp.max(cbuf[...])
    cnt8 = (cnt + 7) // 8 * 8
    dump_d = 128 + jax.lax.iota(jnp.int32, L)

    @pl.loop(0, 8)
    def _(k):
        g = cnt + k

        @pl.when(g < cnt8)
        def _():
            gr = g // 8
            go = lax.rem(g, 8) * L
            ldst[gr, pl.ds(go, L)] = dump_d
            lsrc[gr, pl.ds(go, L)] = g * L + jax.lax.iota(jnp.int32, L)

    plsc.subcore_barrier()
    nblk = cnt8 // 8

    @pl.loop(0, CAPB)
    def _(g8):
        @pl.when(g8 < nblk)
        def _():
            pltpu.async_copy(h3_hbm.at[lsrc.at[g8]], rows, sem).wait()
            pltpu.sync_copy(rows, acc.at[ldst.at[g8]], add=True)

    plsc.subcore_barrier()

    @pl.when(s < 8)
    def _():
        pltpu.sync_copy(acc.at[pl.ds(s * 16, 16)],
                        out_hbm.at[c, pl.ds(s * 16, 16)])


_sc_layer4 = pl.kernel(
    _sc_layer4_body,
    out_type=[jax.ShapeDtypeStruct((NC, 128, 64), jnp.float32)],
    mesh=_mesh,
    scratch_types=[
        pltpu.VMEM_SHARED((ACC4, 64), jnp.float32),
        pltpu.VMEM((CAPB, CHUNK), jnp.int32),
        pltpu.VMEM((CAPB, CHUNK), jnp.int32),
        pltpu.VMEM((CHUNK, 64), jnp.float32),
        pltpu.VMEM((ACC4 // NS, 64), jnp.float32),
        pltpu.VMEM((L,), jnp.int32),
        pltpu.SemaphoreType.DMA,
    ],
    compiler_params=_sc_params)


def _gin_dense(hp, a0, a1, wt, br):
    din = hp.shape[1]
    dout = wt.shape[1]
    bn = 2000

    def body(h_ref, a0_ref, a1_ref, w_ref, b_ref, o_ref):
        x = h_ref[...] + a0_ref[...] + a1_ref[...]
        y = jnp.dot(x, w_ref[...], preferred_element_type=jnp.float32)
        o_ref[...] = jnp.maximum(y + b_ref[...], 0.0)

    return pl.pallas_call(
        body,
        grid=(N // bn,),
        in_specs=[
            pl.BlockSpec((bn, din), lambda i: (i, 0)),
            pl.BlockSpec((bn, din), lambda i: (i, 0)),
            pl.BlockSpec((bn, din), lambda i: (i, 0)),
            pl.BlockSpec((din, dout), lambda i: (0, 0)),
            pl.BlockSpec((1, dout), lambda i: (0, 0)),
        ],
        out_specs=pl.BlockSpec((bn, dout), lambda i: (i, 0)),
        out_shape=jax.ShapeDtypeStruct((N, dout), jnp.float32),
    )(hp, a0, a1, wt, br)


def _gin_dense_split(hp, a0, a1, wt, br):
    din = hp.shape[1]
    bn = 2000

    def body(h_ref, a0_ref, a1_ref, w_ref, b_ref, o_ref):
        x = h_ref[...] + a0_ref[...] + a1_ref[...]
        y = jnp.dot(x, w_ref[...], preferred_element_type=jnp.float32)
        y = jnp.maximum(y + b_ref[...], 0.0)
        o_ref[0] = y[:, 0:16]
        o_ref[1] = y[:, 16:32]

    return pl.pallas_call(
        body,
        grid=(N // bn,),
        in_specs=[
            pl.BlockSpec((bn, din), lambda i: (i, 0)),
            pl.BlockSpec((bn, din), lambda i: (i, 0)),
            pl.BlockSpec((bn, din), lambda i: (i, 0)),
            pl.BlockSpec((din, 32), lambda i: (0, 0)),
            pl.BlockSpec((1, 32), lambda i: (0, 0)),
        ],
        out_specs=pl.BlockSpec((2, bn, 16), lambda i: (0, i, 0)),
        out_shape=jax.ShapeDtypeStruct((2, N, 16), jnp.float32),
    )(hp, a0, a1, wt, br)


def _gin_dense_join(hs, ag, wt, br):
    dout = wt.shape[1]
    bn = 2000

    def body(h_ref, a_ref, w_ref, b_ref, o_ref):
        x = jnp.concatenate([h_ref[0] + a_ref[0],
                             h_ref[1] + a_ref[1]], axis=1)
        y = jnp.dot(x, w_ref[...], preferred_element_type=jnp.float32)
        o_ref[...] = jnp.maximum(y + b_ref[...], 0.0)

    return pl.pallas_call(
        body,
        grid=(N // bn,),
        in_specs=[
            pl.BlockSpec((2, bn, 16), lambda i: (0, i, 0)),
            pl.BlockSpec((2, bn, 16), lambda i: (0, i, 0)),
            pl.BlockSpec((32, dout), lambda i: (0, 0)),
            pl.BlockSpec((1, dout), lambda i: (0, 0)),
        ],
        out_specs=pl.BlockSpec((bn, dout), lambda i: (i, 0)),
        out_shape=jax.ShapeDtypeStruct((N, dout), jnp.float32),
    )(hs, ag, wt, br)


def _tail_body(h3s_ref, a40_ref, a41_ref, w4t_ref, b4_ref,
               wih1_ref, whh1_ref, bih1_ref, bhh1_ref,
               wih2_ref, whh2_ref, bih2_ref, bhh2_ref,
               wih3_ref, whh3_ref, bih3_ref, bhh3_ref,
               wgt_ref, bg_ref, wl1_ref, bl1_ref, wl2_ref, bl2_ref,
               wl3_ref, bl3_ref, o_ref):
    pre = h3s_ref[...] + a40_ref[...] + a41_ref[...]
    h4 = jnp.dot(pre, w4t_ref[...], preferred_element_type=jnp.float32)
    h4 = jnp.maximum(h4 + b4_ref[...], 0.0)
    feat = h4.reshape(16, 8, 128)

    def s2s(wih, whh, bihr, bhhr):
        qs = jnp.zeros((16, 256), jnp.float32)
        hh = jnp.zeros((16, 128), jnp.float32)
        cc = jnp.zeros((16, 128), jnp.float32)
        for _ in range(2):
            gates = (jnp.dot(qs, wih, preferred_element_type=jnp.float32)
                     + bihr
                     + jnp.dot(hh, whh, preferred_element_type=jnp.float32)
                     + bhhr)
            ig = lax.logistic(gates[:, 0:128])
            fg = lax.logistic(gates[:, 128:256])
            gg = jnp.tanh(gates[:, 256:384])
            og = lax.logistic(gates[:, 384:512])
            cc = fg * cc + ig * gg
            hh = og * jnp.tanh(cc)
            e = jnp.sum(feat * hh[:, None, :], axis=2)
            m = jnp.max(e, axis=1, keepdims=True)
            ex = jnp.exp(e - m)
            alpha = ex / jnp.sum(ex, axis=1, keepdims=True)
            r = jnp.sum(alpha[:, :, None] * feat, axis=1)
            qs = jnp.concatenate([hh, r], axis=1)
        return qs

    r1 = s2s(wih1_ref[...], whh1_ref[...], bih1_ref[...], bhh1_ref[...])
    r2 = s2s(wih2_ref[...], whh2_ref[...], bih2_ref[...], bhh2_ref[...])
    r3 = s2s(wih3_ref[...], whh3_ref[...], bih3_ref[...], bhh3_ref[...])
    readout = jnp.concatenate([r1, r2, r3], axis=1)
    t1 = jnp.dot(readout, wgt_ref[...],
                 preferred_element_type=jnp.float32) + bg_ref[0, 0]
    y1 = jnp.tanh(jnp.dot(wl1_ref[...], t1,
                          preferred_element_type=jnp.float32) + bl1_ref[...])
    y2 = jnp.maximum(jnp.dot(wl2_ref[...], y1,
                             preferred_element_type=jnp.float32)
                     + bl2_ref[...], 0.0)
    y3 = jnp.dot(wl3_ref[...], y2,
                 preferred_element_type=jnp.float32) + bl3_ref[...]
    o_ref[...] = lax.logistic(y3)


def _tail(h3s, a40, a41, args):
    return pl.pallas_call(
        _tail_body,
        out_shape=jax.ShapeDtypeStruct((1, 1), jnp.float32),
    )(h3s, a40, a41, *args)


def kernel(h, edge_index, pathway_nodes, W1, b1, W2, b2, W3, b3, W4, b4,
           Wih1, Whh1, bih1, bhh1, Wih2, Whh2, bih2, bhh2,
           Wih3, Whh3, bih3, bhh3, Wg, bg, Wl1, bl1, Wl2, bl2, Wl3, bl3):
    f32 = jnp.float32
    hp = jnp.pad(h, ((0, 0), (0, 16 - h.shape[1])))
    w1t = jnp.pad(W1.T, ((0, 12), (0, 4)))
    b1r = jnp.pad(b1, (0, 4)).reshape(1, 16)
    w2t = jnp.pad(W2.T, ((0, 4), (0, 0)))
    b2r = b2.reshape(1, 32)
    w3t = W3.T
    b3r = b3.reshape(1, 64)

    e3 = edge_index.reshape(2, NCHUNK, CHUNK)
    npad = NCHUNKP - NCHUNK
    pad = jnp.stack([jnp.zeros((npad, CHUNK), jnp.int32),
                     jnp.full((npad, CHUNK), N, jnp.int32)])
    e3 = jnp.concatenate([e3, pad], axis=1)

    agg1, ssrc, sdst, cnts = _sc_agg16c(hp, e3)
    h1 = _gin_dense(hp, agg1[0], agg1[1], w1t, b1r)
    (agg2,) = _sc_agg16(h1, e3)
    h2s = _gin_dense_split(h1, agg2[0], agg2[1], w2t, b2r)
    (agg3,) = _sc_agg32cs(h2s, e3)
    h3 = _gin_dense_join(h2s, agg3, w3t, b3r)
    (agg4,) = _sc_layer4(h3, ssrc, sdst, cnts)

    tail_args = (
        W4.T.astype(f32), b4.reshape(1, 128),
        Wih1.T, Whh1.T, bih1.reshape(1, 512), bhh1.reshape(1, 512),
        Wih2.T, Whh2.T, bih2.reshape(1, 512), bhh2.reshape(1, 512),
        Wih3.T, Whh3.T, bih3.reshape(1, 512), bhh3.reshape(1, 512),
        Wg.T, bg.reshape(1, 1),
        Wl1, bl1.reshape(256, 1), Wl2, bl2.reshape(64, 1),
        Wl3, bl3.reshape(1, 1),
    )
    return _tail(h3[0:128], agg4[0], agg4[1], tail_args)


# speedup vs baseline: 2.3730x; 1.0884x over previous
import dataclasses

import jax
import jax.numpy as jnp
from jax import lax
from jax.experimental import pallas as pl
from jax.experimental.pallas import tpu as pltpu
from jax.experimental.pallas import tpu_sc as plsc

N = 50000
E = 1600000
NC = 2
NS = 16
NW = NC * NS
L = 16

CHUNK = 128
NCHUNK = E // CHUNK
SSC = 32
NCHUNKP = ((NCHUNK + SSC - 1) // SSC) * SSC
NSS = NCHUNKP // SSC
SSI = (NSS + NW - 1) // NW
NACC = N + 16
ZR = 400
NZCH = N // ZR
ZITERS = (NZCH + NS - 1) // NS
CAPG = 512
CAPB = CAPG // 8
ACC4 = 144

_mesh = plsc.VectorSubcoreMesh(core_axis_name="c", subcore_axis_name="s",
                               num_cores=NC, num_subcores=NS)

_sc_params = pltpu.CompilerParams(needs_layout_passes=False,
                                  use_tc_tiling_on_sc=False)


def _zero_vmem_2d(ref, rows, dp):
    z = jnp.zeros((L,), jnp.float32)

    @pl.loop(0, rows)
    def _(r):
        for j in range(0, dp, L):
            ref[r, pl.ds(j, L)] = z


def _make_sc_agg(dp, compact, ssc, sbc):
    nss = NCHUNKP // ssc
    ssi = (nss + NW - 1) // NW
    nsb = ssc // sbc
    out_type = [jax.ShapeDtypeStruct((NC, N, dp), jnp.float32)]
    if compact:
        out_type += [
            jax.ShapeDtypeStruct((NW, CAPB, CHUNK), jnp.int32),
            jax.ShapeDtypeStruct((NW, CAPB, CHUNK), jnp.int32),
            jax.ShapeDtypeStruct((NW, L), jnp.int32),
        ]
    scratch = [
        pltpu.VMEM_SHARED((NACC, dp), jnp.float32),
        pltpu.VMEM((2, 2, ssc, CHUNK), jnp.int32),
        pltpu.VMEM((2, sbc * CHUNK, dp), jnp.float32),
        pltpu.VMEM((ZR, dp), jnp.float32),
        pltpu.SemaphoreType.DMA,
        pltpu.SemaphoreType.DMA,
    ]
    if compact:
        scratch += [
            pltpu.VMEM((CAPB, CHUNK), jnp.int32),
            pltpu.VMEM((CAPB, CHUNK), jnp.int32),
            pltpu.VMEM((L,), jnp.int32),
            pltpu.SMEM((8,), jnp.int32),
        ]

    def body(h_hbm, e_hbm, *rest):
        if compact:
            (out_hbm, ssrc_hbm, sdst_hbm, cnt_hbm,
             acc, idxb, rows, zbuf, gsem, isem,
             lsrc, ldst, cbuf, cms) = rest
        else:
            out_hbm, acc, idxb, rows, zbuf, gsem, isem = rest
        c = lax.axis_index("c")
        s = lax.axis_index("s")
        wid = s * NC + c

        def idx_copies(ss, p):
            base = ss * ssc
            return (
                pltpu.make_async_copy(
                    e_hbm.at[pl.ds(0, 2), pl.ds(base, ssc)],
                    idxb.at[p], isem),
            )

        def gather_copy(p, ci, rset, j):
            return pltpu.make_async_copy(
                h_hbm.at[idxb.at[p, 0, ci]],
                rows.at[rset, pl.ds(j * CHUNK, CHUNK)], gsem)

        _zero_vmem_2d(zbuf, ZR, dp)
        if compact:
            cms[0] = 0

        s0 = wid

        @pl.when(s0 < nss)
        def _():
            for cp in idx_copies(s0, 0):
                cp.start()

        @pl.loop(0, ZITERS)
        def _(i):
            cid = i * NS + s

            @pl.when(cid < NZCH)
            def _():
                pltpu.sync_copy(zbuf, acc.at[pl.ds(cid * ZR, ZR)])

        plsc.subcore_barrier()

        @pl.loop(0, ssi)
        def _(it):
            ss = it * NW + wid
            p = lax.rem(it, 2)

            @pl.when(ss < nss)
            def _():
                for cp in idx_copies(ss, p):
                    cp.wait()
                ss2 = ss + NW

                @pl.when(ss2 < nss)
                def _():
                    for cp in idx_copies(ss2, 1 - p):
                        cp.start()

                for j in range(sbc):
                    gather_copy(p, j, 0, j).start()
                for sb in range(nsb):
                    cset = sb % 2
                    for j in range(sbc):
                        gather_copy(p, sb * sbc + j, cset, j).wait()
                    if sb < nsb - 1:
                        for j in range(sbc):
                            gather_copy(p, (sb + 1) * sbc + j,
                                        1 - cset, j).start()
                    for j in range(sbc):
                        pltpu.sync_copy(
                            rows.at[cset, pl.ds(j * CHUNK, CHUNK)],
                            acc.at[idxb.at[p, 1, sb * sbc + j]], add=True)

                if compact:
                    @pl.loop(0, ssc)
                    def _(ci):
                        m = idxb[p, 1, ci, pl.ds(0, L)] < 128
                        for g in range(1, CHUNK // L):
                            m = jnp.logical_or(
                                m, idxb[p, 1, ci, pl.ds(g * L, L)] < 128)
                        nhit = jnp.max(plsc.all_reduce_population_count(m))

                        @pl.when(nhit > 0)
                        def _():
                            for g in range(CHUNK // L):
                                dv = idxb[p, 1, ci, pl.ds(g * L, L)]
                                mask = dv < 128
                                npos = jnp.max(
                                    plsc.all_reduce_population_count(mask))

                                @pl.when(npos > 0)
                                def _():
                                    gc = cms[0]

                                    @pl.when(gc < CAPG)
                                    def _():
                                        sv = idxb[p, 0, ci, pl.ds(g * L, L)]
                                        gr = gc // 8
                                        go = lax.rem(gc, 8) * L
                                        iot = jax.lax.iota(jnp.int32, 16)
                                        ldst[gr, pl.ds(go, L)] = (
                                            jnp.where(mask, dv, 128 + iot))
                                        lsrc[gr, pl.ds(go, L)] = (
                                            jnp.where(mask, sv,
                                                      gc * L + iot))
                                        cms[0] = gc + 1

        plsc.subcore_barrier()

        @pl.loop(0, ZITERS)
        def _(i):
            cid = i * NS + s

            @pl.when(cid < NZCH)
            def _():
                pltpu.sync_copy(acc.at[pl.ds(cid * ZR, ZR)],
                                out_hbm.at[c, pl.ds(cid * ZR, ZR)])

        if compact:
            pltpu.sync_copy(lsrc, ssrc_hbm.at[wid])
            pltpu.sync_copy(ldst, sdst_hbm.at[wid])
            cbuf[...] = jnp.zeros((L,), jnp.int32) + cms[0]
            pltpu.sync_copy(cbuf, cnt_hbm.at[wid])

    return pl.kernel(body, out_type=out_type, mesh=_mesh,
                     scratch_types=scratch, compiler_params=_sc_params)


_sc_agg16c = _make_sc_agg(16, True, 32, 8)
_sc_agg16 = _make_sc_agg(16, False, 32, 8)


def _make_sc_agg_colsplit(ssc, sbc):
    nss = NCHUNKP // ssc
    ssi = (nss + NS - 1) // NS
    nsb = ssc // sbc
    dp = 16

    def body(h_hbm, e_hbm, out_hbm,
             acc, idxb, rows, zbuf, gsem, isem):
        c = lax.axis_index("c")
        s = lax.axis_index("s")

        def idx_copies(ss, p):
            base = ss * ssc
            return (
                pltpu.make_async_copy(
                    e_hbm.at[pl.ds(0, 2), pl.ds(base, ssc)],
                    idxb.at[p], isem),
            )

        def gather_copy(p, ci, rset, j):
            return pltpu.make_async_copy(
                h_hbm.at[c].at[idxb.at[p, 0, ci]],
                rows.at[rset, pl.ds(j * CHUNK, CHUNK)], gsem)

        _zero_vmem_2d(zbuf, ZR, dp)

        @pl.when(s < nss)
        def _():
            for cp in idx_copies(s, 0):
                cp.start()

        @pl.loop(0, ZITERS)
        def _(i):
            cid = i * NS + s

            @pl.when(cid < NZCH)
            def _():
                pltpu.sync_copy(zbuf, acc.at[pl.ds(cid * ZR, ZR)])

        plsc.subcore_barrier()

        @pl.loop(0, ssi)
        def _(it):
            ss = it * NS + s
            p = lax.rem(it, 2)

            @pl.when(ss < nss)
            def _():
                for cp in idx_copies(ss, p):
                    cp.wait()
                ss2 = ss + NS

                @pl.when(ss2 < nss)
                def _():
                    for cp in idx_copies(ss2, 1 - p):
                        cp.start()

                for j in range(sbc):
                    gather_copy(p, j, 0, j).start()
                for sb in range(nsb):
                    cset = sb % 2
                    for j in range(sbc):
                        gather_copy(p, sb * sbc + j, cset, j).wait()
                    if sb < nsb - 1:
                        for j in range(sbc):
                            gather_copy(p, (sb + 1) * sbc + j,
                                        1 - cset, j).start()
                    for j in range(sbc):
                        pltpu.sync_copy(
                            rows.at[cset, pl.ds(j * CHUNK, CHUNK)],
                            acc.at[idxb.at[p, 1, sb * sbc + j]], add=True)

        plsc.subcore_barrier()

        @pl.loop(0, ZITERS)
        def _(i):
            cid = i * NS + s

            @pl.when(cid < NZCH)
            def _():
                pltpu.sync_copy(acc.at[pl.ds(cid * ZR, ZR)],
                                out_hbm.at[c, pl.ds(cid * ZR, ZR)])

    return pl.kernel(
        body,
        out_type=[jax.ShapeDtypeStruct((NC, N, dp), jnp.float32)],
        mesh=_mesh,
        scratch_types=[
            pltpu.VMEM_SHARED((NACC, dp), jnp.float32),
            pltpu.VMEM((2, 2, ssc, CHUNK), jnp.int32),
            pltpu.VMEM((2, sbc * CHUNK, dp), jnp.float32),
            pltpu.VMEM((ZR, dp), jnp.float32),
            pltpu.SemaphoreType.DMA,
            pltpu.SemaphoreType.DMA,
        ],
        compiler_params=_sc_params)


_sc_agg32cs = _make_sc_agg_colsplit(32, 8)


def _sc_layer4_body(h3_hbm, ssrc_hbm, sdst_hbm, cnt_hbm, out_hbm,
                    acc, lsrc, ldst, rows, zbuf, cbuf, sem):
    c = lax.axis_index("c")
    s = lax.axis_index("s")
    wid = s * NC + c

    _zero_vmem_2d(zbuf, ACC4 // NS, 64)
    pltpu.sync_copy(zbuf, acc.at[pl.ds(s * (ACC4 // NS), ACC4 // NS)])
    pltpu.sync_copy(ssrc_hbm.at[wid], lsrc)
    pltpu.sync_copy(sdst_hbm.at[wid], ldst)
    pltpu.sync_copy(cnt_hbm.at[wid], cbuf)
    cnt = jnp.max(cbuf[...])
    cnt8 = (cnt + 7) // 8 * 8
    dump_d = 128 + jax.lax.iota(jnp.int32, L)

    @pl.loop(0, 8)
    def _(k):
        g = cnt + k

        @pl.when(g < cnt8)
        def _():
            gr = g // 8
            go = lax.rem(g, 8) * L
            ldst[gr, pl.ds(go, L)] = dump_d
            lsrc[gr, pl.ds(go, L)] = g * L + jax.lax.iota(jnp.int32, L)

    plsc.subcore_barrier()
    nblk = cnt8 // 8

    @pl.loop(0, CAPB)
    def _(g8):
        @pl.when(g8 < nblk)
        def _():
            pltpu.async_copy(h3_hbm.at[lsrc.at[g8]], rows, sem).wait()
            pltpu.sync_copy(rows, acc.at[ldst.at[g8]], add=True)

    plsc.subcore_barrier()

    @pl.when(s < 8)
    def _():
        pltpu.sync_copy(acc.at[pl.ds(s * 16, 16)],
                        out_hbm.at[c, pl.ds(s * 16, 16)])


_sc_layer4 = pl.kernel(
    _sc_layer4_body,
    out_type=[jax.ShapeDtypeStruct((NC, 128, 64), jnp.float32)],
    mesh=_mesh,
    scratch_types=[
        pltpu.VMEM_SHARED((ACC4, 64), jnp.float32),
        pltpu.VMEM((CAPB, CHUNK), jnp.int32),
        pltpu.VMEM((CAPB, CHUNK), jnp.int32),
        pltpu.VMEM((CHUNK, 64), jnp.float32),
        pltpu.VMEM((ACC4 // NS, 64), jnp.float32),
        pltpu.VMEM((L,), jnp.int32),
        pltpu.SemaphoreType.DMA,
    ],
    compiler_params=_sc_params)


def _gin_dense(hp, a0, a1, wt, br):
    din = hp.shape[1]
    dout = wt.shape[1]
    bn = 2000

    def body(h_ref, a0_ref, a1_ref, w_ref, b_ref, o_ref):
        x = h_ref[...] + a0_ref[...] + a1_ref[...]
        y = jnp.dot(x, w_ref[...], preferred_element_type=jnp.float32)
        o_ref[...] = jnp.maximum(y + b_ref[...], 0.0)

    return pl.pallas_call(
        body,
        grid=(N // bn,),
        in_specs=[
            pl.BlockSpec((bn, din), lambda i: (i, 0)),
            pl.BlockSpec((bn, din), lambda i: (i, 0)),
            pl.BlockSpec((bn, din), lambda i: (i, 0)),
            pl.BlockSpec((din, dout), lambda i: (0, 0)),
            pl.BlockSpec((1, dout), lambda i: (0, 0)),
        ],
        out_specs=pl.BlockSpec((bn, dout), lambda i: (i, 0)),
        out_shape=jax.ShapeDtypeStruct((N, dout), jnp.float32),
    )(hp, a0, a1, wt, br)


def _gin_dense_split(hp, a0, a1, wt, br):
    din = hp.shape[1]
    bn = 2000

    def body(h_ref, a0_ref, a1_ref, w_ref, b_ref, o_ref):
        x = h_ref[...] + a0_ref[...] + a1_ref[...]
        y = jnp.dot(x, w_ref[...], preferred_element_type=jnp.float32)
        y = jnp.maximum(y + b_ref[...], 0.0)
        o_ref[0] = y[:, 0:16]
        o_ref[1] = y[:, 16:32]

    return pl.pallas_call(
        body,
        grid=(N // bn,),
        in_specs=[
            pl.BlockSpec((bn, din), lambda i: (i, 0)),
            pl.BlockSpec((bn, din), lambda i: (i, 0)),
            pl.BlockSpec((bn, din), lambda i: (i, 0)),
            pl.BlockSpec((din, 32), lambda i: (0, 0)),
            pl.BlockSpec((1, 32), lambda i: (0, 0)),
        ],
        out_specs=pl.BlockSpec((2, bn, 16), lambda i: (0, i, 0)),
        out_shape=jax.ShapeDtypeStruct((2, N, 16), jnp.float32),
    )(hp, a0, a1, wt, br)


def _gin_dense_join(hs, ag, wt, br):
    dout = wt.shape[1]
    bn = 2000

    def body(h_ref, a_ref, w_ref, b_ref, o_ref):
        x = jnp.concatenate([h_ref[0] + a_ref[0],
                             h_ref[1] + a_ref[1]], axis=1)
        y = jnp.dot(x, w_ref[...], preferred_element_type=jnp.float32)
        o_ref[...] = jnp.maximum(y + b_ref[...], 0.0)

    return pl.pallas_call(
        body,
        grid=(N // bn,),
        in_specs=[
            pl.BlockSpec((2, bn, 16), lambda i: (0, i, 0)),
            pl.BlockSpec((2, bn, 16), lambda i: (0, i, 0)),
            pl.BlockSpec((32, dout), lambda i: (0, 0)),
            pl.BlockSpec((1, dout), lambda i: (0, 0)),
        ],
        out_specs=pl.BlockSpec((bn, dout), lambda i: (i, 0)),
        out_shape=jax.ShapeDtypeStruct((N, dout), jnp.float32),
    )(hs, ag, wt, br)


def _tail_body(h3s_ref, a40_ref, a41_ref, w4t_ref, b4_ref,
               wih1_ref, whh1_ref, bih1_ref, bhh1_ref,
               wih2_ref, whh2_ref, bih2_ref, bhh2_ref,
               wih3_ref, whh3_ref, bih3_ref, bhh3_ref,
               wgt_ref, bg_ref, wl1_ref, bl1_ref, wl2_ref, bl2_ref,
               wl3_ref, bl3_ref, o_ref):
    pre = h3s_ref[...] + a40_ref[...] + a41_ref[...]
    h4 = jnp.dot(pre, w4t_ref[...], preferred_element_type=jnp.float32)
    h4 = jnp.maximum(h4 + b4_ref[...], 0.0)
    feat = h4.reshape(16, 8, 128)

    def s2s(wih, whh, bihr, bhhr):
        qs = jnp.zeros((16, 256), jnp.float32)
        hh = jnp.zeros((16, 128), jnp.float32)
        cc = jnp.zeros((16, 128), jnp.float32)
        for _ in range(2):
            gates = (jnp.dot(qs, wih, preferred_element_type=jnp.float32)
                     + bihr
                     + jnp.dot(hh, whh, preferred_element_type=jnp.float32)
                     + bhhr)
            ig = lax.logistic(gates[:, 0:128])
            fg = lax.logistic(gates[:, 128:256])
            gg = jnp.tanh(gates[:, 256:384])
            og = lax.logistic(gates[:, 384:512])
            cc = fg * cc + ig * gg
            hh = og * jnp.tanh(cc)
            e = jnp.sum(feat * hh[:, None, :], axis=2)
            m = jnp.max(e, axis=1, keepdims=True)
            ex = jnp.exp(e - m)
            alpha = ex / jnp.sum(ex, axis=1, keepdims=True)
            r = jnp.sum(alpha[:, :, None] * feat, axis=1)
            qs = jnp.concatenate([hh, r], axis=1)
        return qs

    r1 = s2s(wih1_ref[...], whh1_ref[...], bih1_ref[...], bhh1_ref[...])
    r2 = s2s(wih2_ref[...], whh2_ref[...], bih2_ref[...], bhh2_ref[...])
    r3 = s2s(wih3_ref[...], whh3_ref[...], bih3_ref[...], bhh3_ref[...])
    readout = jnp.concatenate([r1, r2, r3], axis=1)
    t1 = jnp.dot(readout, wgt_ref[...],
                 preferred_element_type=jnp.float32) + bg_ref[0, 0]
    y1 = jnp.tanh(jnp.dot(wl1_ref[...], t1,
                          preferred_element_type=jnp.float32) + bl1_ref[...])
    y2 = jnp.maximum(jnp.dot(wl2_ref[...], y1,
                             preferred_element_type=jnp.float32)
                     + bl2_ref[...], 0.0)
    y3 = jnp.dot(wl3_ref[...], y2,
                 preferred_element_type=jnp.float32) + bl3_ref[...]
    o_ref[...] = lax.logistic(y3)


def _tail(h3s, a40, a41, args):
    return pl.pallas_call(
        _tail_body,
        out_shape=jax.ShapeDtypeStruct((1, 1), jnp.float32),
    )(h3s, a40, a41, *args)


def kernel(h, edge_index, pathway_nodes, W1, b1, W2, b2, W3, b3, W4, b4,
           Wih1, Whh1, bih1, bhh1, Wih2, Whh2, bih2, bhh2,
           Wih3, Whh3, bih3, bhh3, Wg, bg, Wl1, bl1, Wl2, bl2, Wl3, bl3):
    f32 = jnp.float32
    hp = jnp.pad(h, ((0, 0), (0, 16 - h.shape[1])))
    w1t = jnp.pad(W1.T, ((0, 12), (0, 4)))
    b1r = jnp.pad(b1, (0, 4)).reshape(1, 16)
    w2t = jnp.pad(W2.T, ((0, 4), (0, 0)))
    b2r = b2.reshape(1, 32)
    w3t = W3.T
    b3r = b3.reshape(1, 64)

    e3 = edge_index.reshape(2, NCHUNK, CHUNK)
    npad = NCHUNKP - NCHUNK
    ar = jnp.arange(npad * CHUNK, dtype=jnp.int32)
    pad = jnp.stack([(ar % 2048).reshape(npad, CHUNK),
                     N + (ar % 16).reshape(npad, CHUNK)])
    e3 = jnp.concatenate([e3, pad], axis=1)

    agg1, ssrc, sdst, cnts = _sc_agg16c(hp, e3)
    h1 = _gin_dense(hp, agg1[0], agg1[1], w1t, b1r)
    (agg2,) = _sc_agg16(h1, e3)
    h2s = _gin_dense_split(h1, agg2[0], agg2[1], w2t, b2r)
    (agg3,) = _sc_agg32cs(h2s, e3)
    h3 = _gin_dense_join(h2s, agg3, w3t, b3r)
    (agg4,) = _sc_layer4(h3, ssrc, sdst, cnts)

    tail_args = (
        W4.T.astype(f32), b4.reshape(1, 128),
        Wih1.T, Whh1.T, bih1.reshape(1, 512), bhh1.reshape(1, 512),
        Wih2.T, Whh2.T, bih2.reshape(1, 512), bhh2.reshape(1, 512),
        Wih3.T, Whh3.T, bih3.reshape(1, 512), bhh3.reshape(1, 512),
        Wg.T, bg.reshape(1, 1),
        Wl1, bl1.reshape(256, 1), Wl2, bl2.reshape(64, 1),
        Wl3, bl3.reshape(1, 1),
    )
    return _tail(h3[0:128], agg4[0], agg4[1], tail_args)
